# Initial kernel scaffold; baseline (speedup 1.0000x reference)
import jax
import jax.numpy as jnp
from jax.experimental import pallas as pl


def kernel(edge_tensor, bond_W, stereo_W, conj_W, ring_W):
    raise NotImplementedError("write your pallas kernel here")


# speedup vs baseline: 2.1463x; 2.1463x over previous
import jax
import jax.numpy as jnp
from jax import lax
from jax.experimental import pallas as pl
from jax.experimental.pallas import tpu as pltpu
from jax.experimental.pallas import tpu_sc as plsc

N_EDGES = 800000
D = 64
L = 16
NC = 2
NS = 16
NW = NC * NS
C = 640
NCHUNK = N_EDGES // C
TPW = -(-NCHUNK // NW)


def _sc_body(et_hbm, bond_hbm, stereo_hbm, conj_hbm, ring_hbm, out_hbm,
             tab15_v, bs_v, cr_v, ctab_v, idx_v, out_v):
    wid = lax.axis_index("s") * NC + lax.axis_index("c")

    pltpu.sync_copy(bond_hbm, tab15_v.at[pl.ds(0 * D, 5 * D)])
    pltpu.sync_copy(stereo_hbm, tab15_v.at[pl.ds(5 * D, 6 * D)])
    pltpu.sync_copy(conj_hbm, tab15_v.at[pl.ds(11 * D, 2 * D)])
    pltpu.sync_copy(ring_hbm, tab15_v.at[pl.ds(13 * D, 2 * D)])

    def bs_row(r2, _):
        i = r2 // 6
        j = r2 % 6
        for cg in range(D // L):
            o = cg * L
            bs_v[pl.ds(r2 * D + o, L)] = (tab15_v[pl.ds(i * D + o, L)]
                                          + tab15_v[pl.ds((5 + j) * D + o, L)])
        return 0

    lax.fori_loop(0, 30, bs_row, 0)

    for q in range(4):
        for cg in range(D // L):
            o = cg * L
            cr_v[pl.ds(q * D + o, L)] = (tab15_v[pl.ds((11 + q // 2) * D + o, L)]
                                         + tab15_v[pl.ds((13 + q % 2) * D + o, L)])

    def ctab_row(r, _):
        r2 = r // 4
        q = r % 4
        for cg in range(D // L):
            o = cg * L
            ctab_v[pl.ds(r * D + o, L)] = (bs_v[pl.ds(r2 * D + o, L)]
                                           + cr_v[pl.ds(q * D + o, L)])
        return 0

    lax.fori_loop(0, 120, ctab_row, 0)

    lanes = lax.iota(jnp.int32, L)

    def do_chunk(cid):
        base = cid * C
        pltpu.sync_copy(et_hbm.at[pl.ds(base * 4, C * 4)], idx_v)

        def group(g, _):
            e_off = (g * L + lanes) * 4
            i0 = plsc.load_gather(idx_v, [e_off])
            i1 = plsc.load_gather(idx_v, [e_off + 1])
            i2 = plsc.load_gather(idx_v, [e_off + 2])
            i3 = plsc.load_gather(idx_v, [e_off + 3])
            comb64 = (i0 * 24 + i1 * 4 + i2 * 2 + i3) * D
            row64 = (g * L + lanes) * D
            for col in range(D):
                vals = plsc.load_gather(ctab_v, [comb64 + col])
                plsc.store_scatter(out_v, [row64 + col], vals)
            return 0

        lax.fori_loop(0, C // L, group, 0)
        pltpu.sync_copy(out_v, out_hbm.at[pl.ds(base * D, C * D)])

    def chunk_loop(t, _):
        cid = wid + t * NW
        @pl.when(cid < NCHUNK)
        def _():
            do_chunk(cid)
        return 0

    lax.fori_loop(0, TPW, chunk_loop, 0)


@jax.jit
def _edge_encode(et_flat, bond_flat, stereo_flat, conj_flat, ring_flat):
    mesh = plsc.VectorSubcoreMesh(core_axis_name="c", subcore_axis_name="s")
    k = pl.kernel(
        _sc_body,
        out_type=jax.ShapeDtypeStruct((N_EDGES * D,), jnp.float32),
        mesh=mesh,
        compiler_params=pltpu.CompilerParams(needs_layout_passes=False),
        scratch_types=[
            pltpu.VMEM((16 * D,), jnp.float32),
            pltpu.VMEM((30 * D,), jnp.float32),
            pltpu.VMEM((4 * D,), jnp.float32),
            pltpu.VMEM((120 * D,), jnp.float32),
            pltpu.VMEM((C * 4,), jnp.int32),
            pltpu.VMEM((C * D,), jnp.float32),
        ],
    )
    return k(et_flat, bond_flat, stereo_flat, conj_flat, ring_flat)


def kernel(edge_tensor, bond_W, stereo_W, conj_W, ring_W):
    out = _edge_encode(edge_tensor.astype(jnp.int32).reshape(-1),
                       bond_W.reshape(-1), stereo_W.reshape(-1),
                       conj_W.reshape(-1), ring_W.reshape(-1))
    return out.reshape(N_EDGES, D)
